# baseline (speedup 1.0000x reference)
import functools

import jax
import jax.numpy as jnp
import numpy as np
from jax import lax
from jax.experimental import pallas as pl
from jax.experimental.pallas import tpu as pltpu
from jax.experimental.pallas import tpu_sc as plsc

_ROWS = 8
_COLS = 200_000
_BLK = 512
_FULL_BLKS = _COLS // _BLK
_TAIL = _COLS - _FULL_BLKS * _BLK
_BLKS_PER_W = 13
_ITERS = 14

_LO = np.float32(-4.0)
_IVX = np.float32(1.0) / np.float32(0.05)
_IVY = np.float32(1.0) / np.float32(0.05)
_IVZ = np.float32(1.0) / np.float32(0.1)
_GX, _GY, _GZ = np.float32(160), np.float32(160), np.float32(80)


def _bin_component(v, inv_vs, grid):
    r = (v - _LO) * inv_vs
    ok = (r >= np.float32(0.0)) & (r < grid)
    return r.astype(jnp.int32), ok


def _make_voxelizer():
    mesh = plsc.VectorSubcoreMesh(core_axis_name="c", subcore_axis_name="s")

    @functools.partial(
        pl.kernel,
        out_type=jax.ShapeDtypeStruct((3, _ROWS, _COLS), jnp.int32),
        mesh=mesh,
        scratch_types=(
            [pltpu.VMEM((_ROWS, _BLK), jnp.float32) for _ in range(6)]
            + [pltpu.VMEM((_ROWS, _BLK), jnp.int32) for _ in range(6)]
            + [pltpu.VMEM((_ROWS, _TAIL), jnp.float32) for _ in range(3)]
            + [pltpu.VMEM((_ROWS, _TAIL), jnp.int32) for _ in range(3)]
            + [pltpu.SemaphoreType.DMA for _ in range(4)]
        ),
        compiler_params=pltpu.CompilerParams(needs_layout_passes=False),
    )
    def voxelize(pts_hbm, out_hbm,
                 xin0, yin0, zin0, xin1, yin1, zin1,
                 zo0, yo0, xo0, zo1, yo1, xo1,
                 xt, yt, zt, zot, yot, xot,
                 si0, si1, so0, so1):
        wid = lax.axis_index("s") * 2 + lax.axis_index("c")
        xin = (xin0, xin1)
        yin = (yin0, yin1)
        zin = (zin0, zin1)
        zo = (zo0, zo1)
        yo = (yo0, yo1)
        xo = (xo0, xo1)
        sin = (si0, si1)
        sout = (so0, so1)

        def col_of(i):
            t = jnp.minimum(wid * _BLKS_PER_W + i, _FULL_BLKS - 1)
            return pl.multiple_of(t * _BLK, _BLK)

        def in_copies(k):
            b = k % 2
            col = col_of(k)
            return (
                pltpu.make_async_copy(pts_hbm.at[0, :, pl.ds(col, _BLK)], xin[b], sin[b]),
                pltpu.make_async_copy(pts_hbm.at[1, :, pl.ds(col, _BLK)], yin[b], sin[b]),
                pltpu.make_async_copy(pts_hbm.at[2, :, pl.ds(col, _BLK)], zin[b], sin[b]),
            )

        def out_copies(k):
            b = k % 2
            col = col_of(k)
            return (
                pltpu.make_async_copy(zo[b], out_hbm.at[0, :, pl.ds(col, _BLK)], sout[b]),
                pltpu.make_async_copy(yo[b], out_hbm.at[1, :, pl.ds(col, _BLK)], sout[b]),
                pltpu.make_async_copy(xo[b], out_hbm.at[2, :, pl.ds(col, _BLK)], sout[b]),
            )

        def compute(xi, yi, zi, zoo, yoo, xoo, nvec, per_row):
            @plsc.parallel_loop(0, nvec, unroll=4)
            def vec_body(v):
                r = v // per_row
                s = pl.ds((v % per_row) * 16, 16)
                x = xi[r, s]
                y = yi[r, s]
                z = zi[r, s]
                cx, okx = _bin_component(x, _IVX, _GX)
                cy, oky = _bin_component(y, _IVY, _GY)
                cz, okz = _bin_component(z, _IVZ, _GZ)
                valid = okx & oky & okz
                zoo[r, s] = jnp.where(valid, cz, -1)
                yoo[r, s] = jnp.where(valid, cy, -1)
                xoo[r, s] = jnp.where(valid, cx, -1)

        tail = pl.ds(_FULL_BLKS * _BLK, _TAIL)
        pltpu.sync_copy(pts_hbm.at[0, :, tail], xt)
        pltpu.sync_copy(pts_hbm.at[1, :, tail], yt)
        pltpu.sync_copy(pts_hbm.at[2, :, tail], zt)
        compute(xt, yt, zt, zot, yot, xot, _ROWS * (_TAIL // 16), _TAIL // 16)
        pltpu.sync_copy(zot, out_hbm.at[0, :, tail])
        pltpu.sync_copy(yot, out_hbm.at[1, :, tail])
        pltpu.sync_copy(xot, out_hbm.at[2, :, tail])

        for c in in_copies(0):
            c.start()
        for k in range(_ITERS):
            b = k % 2
            if k + 1 < _ITERS:
                for c in in_copies(k + 1):
                    c.start()
            for c in in_copies(k):
                c.wait()
            if k >= 2:
                for c in out_copies(k - 2):
                    c.wait()
            compute(xin[b], yin[b], zin[b], zo[b], yo[b], xo[b],
                    _ROWS * (_BLK // 16), _BLK // 16)
            for c in out_copies(k):
                c.start()
        for k in (_ITERS - 2, _ITERS - 1):
            for c in out_copies(k):
                c.wait()

    return voxelize


_voxelize = _make_voxelizer()


@jax.jit
def kernel(points):
    out = _voxelize(jnp.transpose(points, (2, 0, 1)))
    return jnp.transpose(out, (1, 2, 0)).reshape(-1, 3)


# speedup vs baseline: 4.3947x; 4.3947x over previous
import functools

import jax
import jax.numpy as jnp
import numpy as np
from jax import lax
from jax.experimental import pallas as pl
from jax.experimental.pallas import tpu as pltpu
from jax.experimental.pallas import tpu_sc as plsc

_ROWS = 8
_COLS = 200_000
_BLK = 512
_FULL_BLKS = _COLS // _BLK
_TAIL = _COLS - _FULL_BLKS * _BLK
_BLKS_PER_W = 13
_ITERS = 14

_LO = np.float32(-4.0)
_IVX = np.float32(1.0) / np.float32(0.05)
_IVY = np.float32(1.0) / np.float32(0.05)
_IVZ = np.float32(1.0) / np.float32(0.1)
_GX, _GY, _GZ = np.float32(160), np.float32(160), np.float32(80)


def _bin_component(v, inv_vs, grid):
    r = (v - _LO) * inv_vs
    ok = (r >= np.float32(0.0)) & (r < grid)
    return r.astype(jnp.int32), ok


def _make_voxelizer():
    mesh = plsc.VectorSubcoreMesh(core_axis_name="c", subcore_axis_name="s")

    @functools.partial(
        pl.kernel,
        out_type=(
            jax.ShapeDtypeStruct((_ROWS, _COLS), jnp.int32),
            jax.ShapeDtypeStruct((_ROWS, _COLS), jnp.int32),
            jax.ShapeDtypeStruct((_ROWS, _COLS), jnp.int32),
        ),
        mesh=mesh,
        scratch_types=(
            [pltpu.VMEM((_ROWS, _BLK), jnp.float32) for _ in range(6)]
            + [pltpu.VMEM((_ROWS, _BLK), jnp.int32) for _ in range(6)]
            + [pltpu.VMEM((_ROWS, _TAIL), jnp.float32) for _ in range(3)]
            + [pltpu.VMEM((_ROWS, _TAIL), jnp.int32) for _ in range(3)]
            + [pltpu.SemaphoreType.DMA for _ in range(4)]
        ),
        compiler_params=pltpu.CompilerParams(needs_layout_passes=False),
    )
    def voxelize(pts_hbm, oz_hbm, oy_hbm, ox_hbm,
                 xin0, yin0, zin0, xin1, yin1, zin1,
                 zo0, yo0, xo0, zo1, yo1, xo1,
                 xt, yt, zt, zot, yot, xot,
                 si0, si1, so0, so1):
        wid = lax.axis_index("s") * 2 + lax.axis_index("c")
        xin = (xin0, xin1)
        yin = (yin0, yin1)
        zin = (zin0, zin1)
        zo = (zo0, zo1)
        yo = (yo0, yo1)
        xo = (xo0, xo1)
        sin = (si0, si1)
        sout = (so0, so1)

        def col_of(i):
            t = jnp.minimum(wid * _BLKS_PER_W + i, _FULL_BLKS - 1)
            return pl.multiple_of(t * _BLK, _BLK)

        def in_copies(k):
            b = k % 2
            col = col_of(k)
            return (
                pltpu.make_async_copy(pts_hbm.at[0, :, pl.ds(col, _BLK)], xin[b], sin[b]),
                pltpu.make_async_copy(pts_hbm.at[1, :, pl.ds(col, _BLK)], yin[b], sin[b]),
                pltpu.make_async_copy(pts_hbm.at[2, :, pl.ds(col, _BLK)], zin[b], sin[b]),
            )

        def out_copies(k):
            b = k % 2
            col = col_of(k)
            return (
                pltpu.make_async_copy(zo[b], oz_hbm.at[:, pl.ds(col, _BLK)], sout[b]),
                pltpu.make_async_copy(yo[b], oy_hbm.at[:, pl.ds(col, _BLK)], sout[b]),
                pltpu.make_async_copy(xo[b], ox_hbm.at[:, pl.ds(col, _BLK)], sout[b]),
            )

        def compute(xi, yi, zi, zoo, yoo, xoo, nvec, per_row):
            @plsc.parallel_loop(0, nvec, unroll=4)
            def vec_body(v):
                r = v // per_row
                s = pl.ds((v % per_row) * 16, 16)
                x = xi[r, s]
                y = yi[r, s]
                z = zi[r, s]
                cx, okx = _bin_component(x, _IVX, _GX)
                cy, oky = _bin_component(y, _IVY, _GY)
                cz, okz = _bin_component(z, _IVZ, _GZ)
                valid = okx & oky & okz
                zoo[r, s] = jnp.where(valid, cz, -1)
                yoo[r, s] = jnp.where(valid, cy, -1)
                xoo[r, s] = jnp.where(valid, cx, -1)

        tail = pl.ds(_FULL_BLKS * _BLK, _TAIL)
        pltpu.sync_copy(pts_hbm.at[0, :, tail], xt)
        pltpu.sync_copy(pts_hbm.at[1, :, tail], yt)
        pltpu.sync_copy(pts_hbm.at[2, :, tail], zt)
        compute(xt, yt, zt, zot, yot, xot, _ROWS * (_TAIL // 16), _TAIL // 16)
        pltpu.sync_copy(zot, oz_hbm.at[:, tail])
        pltpu.sync_copy(yot, oy_hbm.at[:, tail])
        pltpu.sync_copy(xot, ox_hbm.at[:, tail])

        for c in in_copies(0):
            c.start()
        for k in range(_ITERS):
            b = k % 2
            if k + 1 < _ITERS:
                for c in in_copies(k + 1):
                    c.start()
            for c in in_copies(k):
                c.wait()
            if k >= 2:
                for c in out_copies(k - 2):
                    c.wait()
            compute(xin[b], yin[b], zin[b], zo[b], yo[b], xo[b],
                    _ROWS * (_BLK // 16), _BLK // 16)
            for c in out_copies(k):
                c.start()
        for k in (_ITERS - 2, _ITERS - 1):
            for c in out_copies(k):
                c.wait()

    return voxelize


_voxelize = _make_voxelizer()


@jax.jit
def kernel(points):
    oz, oy, ox = _voxelize(jnp.transpose(points, (2, 0, 1)))
    return jnp.stack([oz, oy, ox], axis=-1).reshape(-1, 3)
